# Initial kernel scaffold; baseline (speedup 1.0000x reference)
import jax
import jax.numpy as jnp
from jax.experimental import pallas as pl


def kernel(x, node_embs, W_attn1, W_attn2, W_trans, b_trans, node_usages):
    raise NotImplementedError("write your pallas kernel here")


# speedup vs baseline: 3.6076x; 3.6076x over previous
import functools

import jax
import jax.numpy as jnp
import numpy as np
from jax.experimental import pallas as pl
from jax.experimental.pallas import tpu as pltpu

N_NODES = 16
FANOUT = 4
EPSILON = 0.01
USAGE_BETA = 0.001
D_MODEL = 1024
EMB_DIMS = 128
BATCH_TOKENS = 2048

BB = 1024


def _dispatch_math(n, proj, denom, ad_prev, embsT_ref, wrow_ref, ad_ref):
    BBl = proj.shape[0]
    iota = jax.lax.broadcasted_iota(jnp.int32, (BBl, N_NODES), 1)
    logits = jnp.dot(proj, embsT_ref[...], preferred_element_type=jnp.float32)
    tmask = (iota >= n + 1) & (iota <= n + FANOUT)
    tmf = tmask.astype(jnp.float32)
    lm = jnp.where(tmask, logits, -1e30)
    mx = jnp.max(lm, axis=1, keepdims=True)
    e = jnp.exp(lm - mx) * tmf
    s = jnp.sum(e, axis=1, keepdims=True)
    attn = e / s
    wr = wrow_ref[pl.ds(n, 1), :]
    aw = attn * wr
    attn2 = aw / (jnp.sum(aw, axis=1, keepdims=True) + 1e-9)
    asent = attn2 * denom

    ad_pre = jnp.where(iota == n, 0.0, ad_prev) + asent
    m1 = jnp.max(ad_pre, axis=1, keepdims=True)
    i1 = jnp.min(jnp.where(ad_pre == m1, iota, N_NODES), axis=1, keepdims=True)
    hot1 = iota == i1
    ad2 = jnp.where(hot1, -1.0, ad_pre)
    m2 = jnp.max(ad2, axis=1, keepdims=True)
    i2 = jnp.min(jnp.where(ad2 == m2, iota, N_NODES), axis=1, keepdims=True)
    hot2 = iota == i2
    maskf = jnp.where((hot1 | hot2) & (ad_pre > EPSILON), 1.0, 0.0)
    kept = ad_pre * maskf
    scale = 1.0 / (jnp.sum(kept, axis=1, keepdims=True) + 1e-9)
    nf = maskf * scale
    ad_ref[...] = ad_pre * nf
    return asent, nf


def _stage_kernel(x_ref, wa1_ref, wa2_ref, wt_ref, bt_ref, embsT_ref, wrow_ref,
                  out_ref, ring0, ring1, ring2, ring3, coef0, coef1, coef2,
                  coef3, ad_ref):
    n = pl.program_id(1)
    rings = (ring0, ring1, ring2, ring3)
    coefs = (coef0, coef1, coef2, coef3)

    @pl.when(n == 0)
    def _():
        xb = x_ref[...]
        ring0[...] = xb
        ring1[...] = jnp.zeros_like(xb)
        ring2[...] = jnp.zeros_like(xb)
        ring3[...] = jnp.zeros_like(xb)
        proj = jnp.dot(xb, wa1_ref[...], preferred_element_type=jnp.float32)
        ones = jnp.ones((xb.shape[0], 1), jnp.float32)
        zeros = jnp.zeros((xb.shape[0], N_NODES), jnp.float32)
        asent, nf = _dispatch_math(0, proj, ones, zeros, embsT_ref, wrow_ref,
                                   ad_ref)
        coef0[...] = asent * nf
        coef1[...] = jnp.zeros_like(asent)
        coef2[...] = jnp.zeros_like(asent)
        coef3[...] = jnp.zeros_like(asent)

    @pl.when(n > 0)
    def _():
        BBl = out_ref.shape[0]
        iota = jax.lax.broadcasted_iota(jnp.int32, (BBl, N_NODES), 1)
        hot_n = (iota == n).astype(jnp.float32)
        cj = [jnp.sum(coefs[j][...] * hot_n, axis=1, keepdims=True)
              for j in range(4)]
        denom = cj[0] + cj[1] + cj[2] + cj[3]
        aggr = (cj[0] * ring0[...] + cj[1] * ring1[...]
                + cj[2] * ring2[...] + cj[3] * ring3[...])
        aggr = aggr / (denom + 1e-9)
        out = jnp.dot(aggr, wt_ref[0], preferred_element_type=jnp.float32)
        out = jnp.maximum(out + bt_ref[pl.ds(n, 1), :], 0.0)
        proj = jnp.dot(out, wa2_ref[...], preferred_element_type=jnp.float32)
        asent, nf = _dispatch_math(n, proj, denom, ad_ref[...], embsT_ref,
                                   wrow_ref, ad_ref)
        slot = jax.lax.rem(n, 4)
        for j in range(4):
            @pl.when(slot == j)
            def _(j=j):
                rings[j][...] = out
                coefs[j][...] = asent * nf

            @pl.when(slot != j)
            def _(j=j):
                coefs[j][...] = coefs[j][...] * nf

        @pl.when(n == N_NODES - 2)
        def _():
            c15 = [coefs[j][:, N_NODES - 1:N_NODES] for j in range(4)]
            den = c15[0] + c15[1] + c15[2] + c15[3]
            outf = (c15[0] * ring0[...] + c15[1] * ring1[...]
                    + c15[2] * ring2[...] + c15[3] * ring3[...])
            out_ref[...] = outf / (den + 1e-9)


@jax.jit
def kernel(x, node_embs, W_attn1, W_attn2, W_trans, b_trans, node_usages):
    B = x.shape[0]
    embsT = node_embs.T
    inv = 1.0 / (node_usages + USAGE_BETA)
    tmask_np = np.zeros((N_NODES, N_NODES), np.float32)
    for nid in range(N_NODES - 1):
        tmask_np[nid, nid + 1:min(nid + 1 + FANOUT, N_NODES)] = 1.0
    tm = jnp.asarray(tmask_np)
    aw = inv[None, :] * tm
    wrow = aw / (jnp.sum(aw, axis=1, keepdims=True) + 1e-30)

    nb = B // BB
    grid = (nb, N_NODES - 1)
    out = pl.pallas_call(
        _stage_kernel,
        grid=grid,
        in_specs=[
            pl.BlockSpec((BB, D_MODEL), lambda b, n: (b, 0)),
            pl.BlockSpec((D_MODEL, EMB_DIMS), lambda b, n: (0, 0)),
            pl.BlockSpec((D_MODEL, EMB_DIMS), lambda b, n: (0, 0)),
            pl.BlockSpec((1, D_MODEL, D_MODEL), lambda b, n: (n, 0, 0)),
            pl.BlockSpec((N_NODES, D_MODEL), lambda b, n: (0, 0)),
            pl.BlockSpec((EMB_DIMS, N_NODES), lambda b, n: (0, 0)),
            pl.BlockSpec((N_NODES, N_NODES), lambda b, n: (0, 0)),
        ],
        out_specs=pl.BlockSpec((BB, D_MODEL), lambda b, n: (b, 0)),
        out_shape=jax.ShapeDtypeStruct((B, D_MODEL), jnp.float32),
        scratch_shapes=[
            pltpu.VMEM((BB, D_MODEL), jnp.float32),
            pltpu.VMEM((BB, D_MODEL), jnp.float32),
            pltpu.VMEM((BB, D_MODEL), jnp.float32),
            pltpu.VMEM((BB, D_MODEL), jnp.float32),
            pltpu.VMEM((BB, N_NODES), jnp.float32),
            pltpu.VMEM((BB, N_NODES), jnp.float32),
            pltpu.VMEM((BB, N_NODES), jnp.float32),
            pltpu.VMEM((BB, N_NODES), jnp.float32),
            pltpu.VMEM((BB, N_NODES), jnp.float32),
        ],
        compiler_params=pltpu.CompilerParams(
            dimension_semantics=("arbitrary", "arbitrary"),
        ),
    )(x, W_attn1, W_attn2, W_trans, b_trans, embsT, wrow)
    return out
